# baseline (speedup 1.0000x reference)
import functools
import jax
import jax.numpy as jnp
from jax import lax
from jax.experimental import pallas as pl
from jax.experimental.pallas import tpu as pltpu
from jax.experimental.pallas import tpu_sc as plsc

_MAX_POS = 8192
_HIDDEN = 1024
_BATCH = 4
_CH = 32
_NBUF = 2


def _sc_body(tab_hbm, out_hbm, bufs, in_sem, out_sem):
    info = plsc.get_sparse_core_info()
    nc, ns = info.num_cores, info.num_subcores
    nw = nc * ns
    wid = lax.axis_index("s") * nc + lax.axis_index("c")
    rows_per_w = _MAX_POS // nw
    nchunk = rows_per_w // _CH
    row0 = wid * rows_per_w

    def in_copy(i):
        return pltpu.make_async_copy(
            tab_hbm.at[pl.ds(row0 + i * _CH, _CH), :],
            bufs.at[i % _NBUF],
            in_sem.at[i % _NBUF],
        )

    def out_copy(i, b):
        return pltpu.make_async_copy(
            bufs.at[i % _NBUF],
            out_hbm.at[b, pl.ds(row0 + i * _CH, _CH), :],
            out_sem.at[i % _NBUF, b],
        )

    in_copy(0).start()
    for k in range(nchunk):
        if k + 1 < nchunk:
            if k + 1 - _NBUF >= 0:
                for b in range(_BATCH):
                    out_copy(k + 1 - _NBUF, b).wait()
            in_copy(k + 1).start()
        in_copy(k).wait()
        for b in range(_BATCH):
            out_copy(k, b).start()
    for i in range(max(0, nchunk - _NBUF), nchunk):
        for b in range(_BATCH):
            out_copy(i, b).wait()


def kernel(input_ids, table):
    batch, seq = input_ids.shape
    hidden = table.shape[1]
    mesh = plsc.VectorSubcoreMesh(core_axis_name="c", subcore_axis_name="s")
    k = functools.partial(
        pl.kernel,
        mesh=mesh,
        out_type=jax.ShapeDtypeStruct((batch, seq, hidden), table.dtype),
        scratch_types=[
            pltpu.VMEM((_NBUF, _CH, hidden), table.dtype),
            pltpu.SemaphoreType.DMA((_NBUF,)),
            pltpu.SemaphoreType.DMA((_NBUF, _BATCH)),
        ],
    )(_sc_body)
    return k(table)


# speedup vs baseline: 1.4955x; 1.4955x over previous
import jax
import jax.numpy as jnp
from jax.experimental import pallas as pl
from jax.experimental.pallas import tpu as pltpu

_BLK = 1024


def _bcast_pipeline(tab_ref, out_ref, bufs, in_sem, out_sem):
    nblk = tab_ref.shape[0] // _BLK
    batch = out_ref.shape[0]

    def in_copy(i):
        return pltpu.make_async_copy(
            tab_ref.at[pl.ds(i * _BLK, _BLK), :], bufs.at[i], in_sem.at[i]
        )

    def out_copy(i, b):
        return pltpu.make_async_copy(
            bufs.at[i], out_ref.at[b, pl.ds(i * _BLK, _BLK), :], out_sem.at[i, b]
        )

    for i in range(nblk):
        in_copy(i).start()
    for i in range(nblk):
        in_copy(i).wait()
        for b in range(batch):
            out_copy(i, b).start()
    for i in range(nblk):
        for b in range(batch):
            out_copy(i, b).wait()


def kernel(input_ids, table):
    batch, seq = input_ids.shape
    hidden = table.shape[1]
    nblk = seq // _BLK
    out = pl.pallas_call(
        _bcast_pipeline,
        in_specs=[pl.BlockSpec(memory_space=pl.ANY)],
        out_specs=pl.BlockSpec(memory_space=pl.ANY),
        out_shape=jax.ShapeDtypeStruct((batch, seq, hidden), table.dtype),
        scratch_shapes=[
            pltpu.VMEM((nblk, _BLK, hidden), table.dtype),
            pltpu.SemaphoreType.DMA((nblk,)),
            pltpu.SemaphoreType.DMA((nblk, 4)),
        ],
    )(table)
    return out


# speedup vs baseline: 1.5144x; 1.0126x over previous
import jax
import jax.numpy as jnp
from jax.experimental import pallas as pl
from jax.experimental.pallas import tpu as pltpu

_BLK = 1024
_NBUF = 8


def _pipelined_bcast(tab_ref, out_ref, bufs, in_sem, out_sem):
    nblk = tab_ref.shape[0] // _BLK
    batch = out_ref.shape[0]

    def in_copy(i):
        return pltpu.make_async_copy(
            tab_ref.at[pl.ds(i * _BLK, _BLK), :],
            bufs.at[i % _NBUF],
            in_sem.at[i % _NBUF],
        )

    def out_copy(i, b):
        return pltpu.make_async_copy(
            bufs.at[i % _NBUF],
            out_ref.at[b, pl.ds(i * _BLK, _BLK), :],
            out_sem.at[i % _NBUF, b],
        )

    in_copy(0).start()
    in_copy(1).start()
    for k in range(nblk):
        if k + 2 < nblk:
            if k + 2 - _NBUF >= 0:
                for b in range(batch):
                    out_copy(k + 2 - _NBUF, b).wait()
            in_copy(k + 2).start()
        in_copy(k).wait()
        for b in range(batch):
            out_copy(k, b).start()
    for i in range(max(0, nblk - _NBUF), nblk):
        for b in range(batch):
            out_copy(i, b).wait()


def kernel(input_ids, table):
    batch, seq = input_ids.shape
    hidden = table.shape[1]
    out = pl.pallas_call(
        _pipelined_bcast,
        in_specs=[pl.BlockSpec(memory_space=pl.ANY)],
        out_specs=pl.BlockSpec(memory_space=pl.ANY),
        out_shape=jax.ShapeDtypeStruct((batch, seq, hidden), table.dtype),
        scratch_shapes=[
            pltpu.VMEM((_NBUF, _BLK, hidden), table.dtype),
            pltpu.SemaphoreType.DMA((_NBUF,)),
            pltpu.SemaphoreType.DMA((_NBUF, 4)),
        ],
    )(table)
    return out
